# baseline (device time: 110375 ns/iter reference)
import jax
import jax.numpy as jnp
from jax import lax
from jax.experimental import pallas as pl
from jax.experimental.pallas import tpu as pltpu

N_DEV = 4


def _gelu(z):
    return 0.5 * z * (1.0 + jnp.tanh(0.7978845608 * (z + 0.044715 * z * z * z)))


def kernel(A, B):
    m, k = A.shape
    _, n = B.shape
    mc = m // N_DEV
    nh = n // 2

    def body(a_ref, b_ref, out_ref, a_bf, b_bf,
             cw_comm, ccw_comm, agcw_comm, agccw_comm,
             cw_send, cw_recv, ccw_send, ccw_recv,
             agcw_send, agcw_recv, agccw_send, agccw_recv):
        my = lax.axis_index("i")
        left = (my + N_DEV - 1) % N_DEV
        right = (my + 1) % N_DEV

        barrier_sem = pltpu.get_barrier_semaphore()
        for nbr in (left, right):
            pl.semaphore_signal(
                barrier_sem, inc=1,
                device_id=(nbr,), device_id_type=pl.DeviceIdType.MESH,
            )
        pl.semaphore_wait(barrier_sem, 2)

        def cast_a(c):
            a_bf[pl.ds(c * mc, mc), :] = (
                a_ref[pl.ds(c * mc, mc), :].astype(jnp.bfloat16))

        def cast_b(col0):
            b_bf[:, pl.ds(col0, nh)] = (
                b_ref[:, pl.ds(col0, nh)].astype(jnp.bfloat16))

        def partial(c, col0):
            return jnp.dot(
                a_bf[pl.ds(c * mc, mc), :], b_bf[:, pl.ds(col0, nh)],
                preferred_element_type=jnp.float32,
            )

        def rs_rdma(s, buf, sends, recvs, dev):
            ss = s % 2
            rr = (s + 1) % 2
            return pltpu.make_async_remote_copy(
                src_ref=buf.at[ss], dst_ref=buf.at[rr],
                send_sem=sends.at[ss], recv_sem=recvs.at[rr],
                device_id=(dev,), device_id_type=pl.DeviceIdType.MESH,
            ), rr

        def ag_rdma(t, buf, sends, recvs, dev):
            return pltpu.make_async_remote_copy(
                src_ref=buf.at[t], dst_ref=buf.at[t + 1],
                send_sem=sends.at[t], recv_sem=recvs.at[t],
                device_id=(dev,), device_id_type=pl.DeviceIdType.MESH,
            )

        cast_a(my)
        cast_b(0)
        cw_comm[0] = partial(my, 0).astype(jnp.bfloat16)
        cw0, rr0 = rs_rdma(0, cw_comm, cw_send, cw_recv, right)
        cw0.start()
        cast_b(nh)
        ccw_comm[0] = partial(my, nh).astype(jnp.bfloat16)
        ccw0, _ = rs_rdma(0, ccw_comm, ccw_send, ccw_recv, left)
        ccw0.start()
        for d in range(1, N_DEV):
            cast_a((my + d) % N_DEV)

        rdmas = (cw0, ccw0)
        for s in range(N_DEV - 1):
            cw, ccw = rdmas
            rr = (s + 1) % 2
            p_cw = partial((my - s - 1) % N_DEV, 0)
            p_ccw = partial((my + s + 1) % N_DEV, nh)
            cw.wait()
            cw_comm[rr] = (
                cw_comm[rr].astype(jnp.float32) + p_cw
            ).astype(jnp.bfloat16)
            if s < N_DEV - 2:
                cw_n, _ = rs_rdma(s + 1, cw_comm, cw_send, cw_recv, right)
                cw_n.start()
            ccw.wait()
            ccw_comm[rr] = (
                ccw_comm[rr].astype(jnp.float32) + p_ccw
            ).astype(jnp.bfloat16)
            if s < N_DEV - 2:
                ccw_n, _ = rs_rdma(s + 1, ccw_comm, ccw_send, ccw_recv, left)
                ccw_n.start()
                rdmas = (cw_n, ccw_n)

        last = (N_DEV - 1) % 2
        own_cw = (my + 1) % N_DEV
        own_ccw = (my + N_DEV - 1) % N_DEV
        g_cw = _gelu(cw_comm[last].astype(jnp.float32))
        agcw_comm[0] = g_cw.astype(jnp.bfloat16)
        agcw0 = ag_rdma(0, agcw_comm, agcw_send, agcw_recv, right)
        agcw0.start()
        g_ccw = _gelu(ccw_comm[last].astype(jnp.float32))
        agccw_comm[0] = g_ccw.astype(jnp.bfloat16)
        agccw0 = ag_rdma(0, agccw_comm, agccw_send, agccw_recv, left)
        agccw0.start()
        out_ref[pl.ds(own_cw * mc, mc), pl.ds(0, nh)] = g_cw
        out_ref[pl.ds(own_ccw * mc, mc), pl.ds(nh, nh)] = g_ccw

        for t in range(N_DEV - 1):
            cw = ag_rdma(t, agcw_comm, agcw_send, agcw_recv, right)
            ccw = ag_rdma(t, agccw_comm, agccw_send, agccw_recv, left)
            cw.wait_recv()
            if t < N_DEV - 2:
                ag_rdma(t + 1, agcw_comm, agcw_send, agcw_recv, right).start()
            ccw.wait_recv()
            if t < N_DEV - 2:
                ag_rdma(t + 1, agccw_comm, agccw_send, agccw_recv, left).start()
            out_ref[pl.ds(((my - t) % N_DEV) * mc, mc), pl.ds(0, nh)] = (
                agcw_comm[t + 1].astype(jnp.float32))
            out_ref[pl.ds(((my + t) % N_DEV) * mc, mc), pl.ds(nh, nh)] = (
                agccw_comm[t + 1].astype(jnp.float32))

        for t in range(N_DEV - 1):
            ag_rdma(t, agcw_comm, agcw_send, agcw_recv, right).wait_send()
            ag_rdma(t, agccw_comm, agccw_send, agccw_recv, left).wait_send()

    return pl.pallas_call(
        body,
        out_shape=jax.ShapeDtypeStruct((m, n), jnp.float32),
        in_specs=[
            pl.BlockSpec(memory_space=pltpu.VMEM),
            pl.BlockSpec(memory_space=pltpu.VMEM),
        ],
        out_specs=pl.BlockSpec(memory_space=pltpu.VMEM),
        scratch_shapes=[
            pltpu.VMEM((m, k), jnp.bfloat16),
            pltpu.VMEM((k, n), jnp.bfloat16),
            pltpu.VMEM((2, mc, nh), jnp.bfloat16),
            pltpu.VMEM((2, mc, nh), jnp.bfloat16),
            pltpu.VMEM((N_DEV, mc, nh), jnp.bfloat16),
            pltpu.VMEM((N_DEV, mc, nh), jnp.bfloat16),
            pltpu.SemaphoreType.DMA((2,)),
            pltpu.SemaphoreType.DMA((2,)),
            pltpu.SemaphoreType.DMA((2,)),
            pltpu.SemaphoreType.DMA((2,)),
            pltpu.SemaphoreType.DMA((N_DEV - 1,)),
            pltpu.SemaphoreType.DMA((N_DEV - 1,)),
            pltpu.SemaphoreType.DMA((N_DEV - 1,)),
            pltpu.SemaphoreType.DMA((N_DEV - 1,)),
        ],
        compiler_params=pltpu.CompilerParams(
            collective_id=0,
            vmem_limit_bytes=100 * 1024 * 1024,
        ),
    )(A, B)


# device time: 105841 ns/iter; 1.0428x vs baseline; 1.0428x over previous
import jax
import jax.numpy as jnp
from jax import lax
from jax.experimental import pallas as pl
from jax.experimental.pallas import tpu as pltpu

N_DEV = 4


def _gelu(z):
    return 0.5 * z * (1.0 + jnp.tanh(0.7978845608 * (z + 0.044715 * z * z * z)))


def kernel(A, B):
    m, k = A.shape
    _, n = B.shape
    mc = m // N_DEV
    nh = n // 2

    def body(a_ref, b_ref, out_ref,
             cw_comm, ccw_comm, agcw_comm, agccw_comm, stage,
             cw_send, cw_recv, ccw_send, ccw_recv,
             agcw_send, agcw_recv, agccw_send, agccw_recv, out_sems):
        my = lax.axis_index("i")
        left = (my + N_DEV - 1) % N_DEV
        right = (my + 1) % N_DEV

        barrier_sem = pltpu.get_barrier_semaphore()
        for nbr in (left, right):
            pl.semaphore_signal(
                barrier_sem, inc=1,
                device_id=(nbr,), device_id_type=pl.DeviceIdType.MESH,
            )
        pl.semaphore_wait(barrier_sem, 2)

        def partial(c, col0):
            return jnp.dot(
                a_ref[pl.ds(c * mc, mc), :], b_ref[:, pl.ds(col0, nh)],
                preferred_element_type=jnp.float32,
            )

        def rs_rdma(s, buf, sends, recvs, dev):
            ss = s % 2
            rr = (s + 1) % 2
            return pltpu.make_async_remote_copy(
                src_ref=buf.at[ss], dst_ref=buf.at[rr],
                send_sem=sends.at[ss], recv_sem=recvs.at[rr],
                device_id=(dev,), device_id_type=pl.DeviceIdType.MESH,
            ), rr

        def ag_rdma(t, buf, sends, recvs, dev):
            return pltpu.make_async_remote_copy(
                src_ref=buf.at[t], dst_ref=buf.at[t + 1],
                send_sem=sends.at[t], recv_sem=recvs.at[t],
                device_id=(dev,), device_id_type=pl.DeviceIdType.MESH,
            )

        def store_out(slot, vals, c, col0):
            stage[slot] = vals
            cp = pltpu.make_async_copy(
                stage.at[slot],
                out_ref.at[pl.ds(c * mc, mc), pl.ds(col0, nh)],
                out_sems.at[slot],
            )
            cp.start()
            return cp

        cw_comm[0] = partial(my, 0).astype(jnp.bfloat16)
        cw0, _ = rs_rdma(0, cw_comm, cw_send, cw_recv, right)
        cw0.start()
        ccw_comm[0] = partial(my, nh).astype(jnp.bfloat16)
        ccw0, _ = rs_rdma(0, ccw_comm, ccw_send, ccw_recv, left)
        ccw0.start()

        rdmas = (cw0, ccw0)
        for s in range(N_DEV - 1):
            cw, ccw = rdmas
            rr = (s + 1) % 2
            p_cw = partial((my - s - 1) % N_DEV, 0)
            p_ccw = partial((my + s + 1) % N_DEV, nh)
            cw.wait()
            cw_comm[rr] = (
                cw_comm[rr].astype(jnp.float32) + p_cw
            ).astype(jnp.bfloat16)
            if s < N_DEV - 2:
                cw_n, _ = rs_rdma(s + 1, cw_comm, cw_send, cw_recv, right)
                cw_n.start()
            ccw.wait()
            ccw_comm[rr] = (
                ccw_comm[rr].astype(jnp.float32) + p_ccw
            ).astype(jnp.bfloat16)
            if s < N_DEV - 2:
                ccw_n, _ = rs_rdma(s + 1, ccw_comm, ccw_send, ccw_recv, left)
                ccw_n.start()
                rdmas = (cw_n, ccw_n)

        last = (N_DEV - 1) % 2
        own_cw = (my + 1) % N_DEV
        own_ccw = (my + N_DEV - 1) % N_DEV
        g_cw = _gelu(cw_comm[last].astype(jnp.float32))
        agcw_comm[0] = g_cw.astype(jnp.bfloat16)
        agcw0 = ag_rdma(0, agcw_comm, agcw_send, agcw_recv, right)
        agcw0.start()
        g_ccw = _gelu(ccw_comm[last].astype(jnp.float32))
        agccw_comm[0] = g_ccw.astype(jnp.bfloat16)
        agccw0 = ag_rdma(0, agccw_comm, agccw_send, agccw_recv, left)
        agccw0.start()
        out_cps = [
            store_out(0, g_cw, own_cw, 0),
            store_out(1, g_ccw, own_ccw, nh),
        ]

        for t in range(N_DEV - 1):
            cw = ag_rdma(t, agcw_comm, agcw_send, agcw_recv, right)
            ccw = ag_rdma(t, agccw_comm, agccw_send, agccw_recv, left)
            cw.wait_recv()
            if t < N_DEV - 2:
                ag_rdma(t + 1, agcw_comm, agcw_send, agcw_recv, right).start()
            ccw.wait_recv()
            if t < N_DEV - 2:
                ag_rdma(t + 1, agccw_comm, agccw_send, agccw_recv, left).start()
            out_cps.append(store_out(
                2 + 2 * t, agcw_comm[t + 1].astype(jnp.float32),
                (my - t) % N_DEV, 0))
            out_cps.append(store_out(
                3 + 2 * t, agccw_comm[t + 1].astype(jnp.float32),
                (my + t) % N_DEV, nh))

        for cp in out_cps:
            cp.wait()
        for t in range(N_DEV - 1):
            ag_rdma(t, agcw_comm, agcw_send, agcw_recv, right).wait_send()
            ag_rdma(t, agccw_comm, agccw_send, agccw_recv, left).wait_send()

    return pl.pallas_call(
        body,
        out_shape=jax.ShapeDtypeStruct((m, n), jnp.float32),
        in_specs=[
            pl.BlockSpec(memory_space=pltpu.VMEM),
            pl.BlockSpec(memory_space=pltpu.VMEM),
        ],
        out_specs=pl.BlockSpec(memory_space=pl.ANY),
        scratch_shapes=[
            pltpu.VMEM((2, mc, nh), jnp.bfloat16),
            pltpu.VMEM((2, mc, nh), jnp.bfloat16),
            pltpu.VMEM((N_DEV, mc, nh), jnp.bfloat16),
            pltpu.VMEM((N_DEV, mc, nh), jnp.bfloat16),
            pltpu.VMEM((8, mc, nh), jnp.float32),
            pltpu.SemaphoreType.DMA((2,)),
            pltpu.SemaphoreType.DMA((2,)),
            pltpu.SemaphoreType.DMA((2,)),
            pltpu.SemaphoreType.DMA((2,)),
            pltpu.SemaphoreType.DMA((N_DEV - 1,)),
            pltpu.SemaphoreType.DMA((N_DEV - 1,)),
            pltpu.SemaphoreType.DMA((N_DEV - 1,)),
            pltpu.SemaphoreType.DMA((N_DEV - 1,)),
            pltpu.SemaphoreType.DMA((8,)),
        ],
        compiler_params=pltpu.CompilerParams(
            collective_id=0,
            vmem_limit_bytes=100 * 1024 * 1024,
        ),
    )(A, B)


# device time: 105020 ns/iter; 1.0510x vs baseline; 1.0078x over previous
import jax
import jax.numpy as jnp
from jax import lax
from jax.experimental import pallas as pl
from jax.experimental.pallas import tpu as pltpu

N_DEV = 4


def _gelu(z):
    return 0.5 * z * (1.0 + jnp.tanh(0.7978845608 * (z + 0.044715 * z * z * z)))


def kernel(A, B):
    m, k = A.shape
    _, n = B.shape
    mc = m // N_DEV
    nh = n // 2

    def body(a_hbm, b_hbm, out_ref,
             a_v, b_v, cw_comm, ccw_comm, agcw_comm, agccw_comm, stage,
             cw_send, cw_recv, ccw_send, ccw_recv,
             agcw_send, agcw_recv, agccw_send, agccw_recv,
             out_sems, in_sems):
        my = lax.axis_index("i")
        left = (my + N_DEV - 1) % N_DEV
        right = (my + 1) % N_DEV

        def load_a(d):
            c = (my + d) % N_DEV
            cp = pltpu.make_async_copy(
                a_hbm.at[pl.ds(c * mc, mc), :],
                a_v.at[pl.ds(c * mc, mc), :],
                in_sems.at[d],
            )
            cp.start()
            return cp

        def load_b(half):
            cp = pltpu.make_async_copy(
                b_hbm.at[:, pl.ds(half * nh, nh)],
                b_v.at[:, pl.ds(half * nh, nh)],
                in_sems.at[N_DEV + half],
            )
            cp.start()
            return cp

        ld_a0 = load_a(0)
        ld_b0 = load_b(0)

        barrier_sem = pltpu.get_barrier_semaphore()
        for nbr in (left, right):
            pl.semaphore_signal(
                barrier_sem, inc=1,
                device_id=(nbr,), device_id_type=pl.DeviceIdType.MESH,
            )
        pl.semaphore_wait(barrier_sem, 2)

        def partial(c, col0):
            return jnp.dot(
                a_v[pl.ds(c * mc, mc), :], b_v[:, pl.ds(col0, nh)],
                preferred_element_type=jnp.float32,
            )

        def rs_rdma(s, buf, sends, recvs, dev):
            ss = s % 2
            rr = (s + 1) % 2
            return pltpu.make_async_remote_copy(
                src_ref=buf.at[ss], dst_ref=buf.at[rr],
                send_sem=sends.at[ss], recv_sem=recvs.at[rr],
                device_id=(dev,), device_id_type=pl.DeviceIdType.MESH,
            ), rr

        def ag_rdma(t, buf, sends, recvs, dev):
            return pltpu.make_async_remote_copy(
                src_ref=buf.at[t], dst_ref=buf.at[t + 1],
                send_sem=sends.at[t], recv_sem=recvs.at[t],
                device_id=(dev,), device_id_type=pl.DeviceIdType.MESH,
            )

        def store_out(slot, vals, c, col0):
            stage[slot] = vals
            cp = pltpu.make_async_copy(
                stage.at[slot],
                out_ref.at[pl.ds(c * mc, mc), pl.ds(col0, nh)],
                out_sems.at[slot],
            )
            cp.start()
            return cp

        ld_a0.wait()
        ld_b0.wait()
        cw_comm[0] = partial(my, 0).astype(jnp.bfloat16)
        cw0, _ = rs_rdma(0, cw_comm, cw_send, cw_recv, right)
        cw0.start()
        ld_b1 = load_b(1)
        ld_rest = {d: load_a(d) for d in range(1, N_DEV)}
        ld_b1.wait()
        ccw_comm[0] = partial(my, nh).astype(jnp.bfloat16)
        ccw0, _ = rs_rdma(0, ccw_comm, ccw_send, ccw_recv, left)
        ccw0.start()

        rdmas = (cw0, ccw0)
        for s in range(N_DEV - 1):
            cw, ccw = rdmas
            rr = (s + 1) % 2
            for d in sorted({s + 1, N_DEV - s - 1}):
                cp = ld_rest.pop(d, None)
                if cp is not None:
                    cp.wait()
            p_cw = partial((my - s - 1) % N_DEV, 0)
            p_ccw = partial((my + s + 1) % N_DEV, nh)
            cw.wait()
            cw_comm[rr] = (
                cw_comm[rr].astype(jnp.float32) + p_cw
            ).astype(jnp.bfloat16)
            if s < N_DEV - 2:
                cw_n, _ = rs_rdma(s + 1, cw_comm, cw_send, cw_recv, right)
                cw_n.start()
            ccw.wait()
            ccw_comm[rr] = (
                ccw_comm[rr].astype(jnp.float32) + p_ccw
            ).astype(jnp.bfloat16)
            if s < N_DEV - 2:
                ccw_n, _ = rs_rdma(s + 1, ccw_comm, ccw_send, ccw_recv, left)
                ccw_n.start()
                rdmas = (cw_n, ccw_n)

        last = (N_DEV - 1) % 2
        own_cw = (my + 1) % N_DEV
        own_ccw = (my + N_DEV - 1) % N_DEV
        g_cw = _gelu(cw_comm[last].astype(jnp.float32))
        agcw_comm[0] = g_cw.astype(jnp.bfloat16)
        agcw0 = ag_rdma(0, agcw_comm, agcw_send, agcw_recv, right)
        agcw0.start()
        g_ccw = _gelu(ccw_comm[last].astype(jnp.float32))
        agccw_comm[0] = g_ccw.astype(jnp.bfloat16)
        agccw0 = ag_rdma(0, agccw_comm, agccw_send, agccw_recv, left)
        agccw0.start()
        out_cps = [
            store_out(0, g_cw, own_cw, 0),
            store_out(1, g_ccw, own_ccw, nh),
        ]

        for t in range(N_DEV - 1):
            cw = ag_rdma(t, agcw_comm, agcw_send, agcw_recv, right)
            ccw = ag_rdma(t, agccw_comm, agccw_send, agccw_recv, left)
            cw.wait_recv()
            if t < N_DEV - 2:
                ag_rdma(t + 1, agcw_comm, agcw_send, agcw_recv, right).start()
            ccw.wait_recv()
            if t < N_DEV - 2:
                ag_rdma(t + 1, agccw_comm, agccw_send, agccw_recv, left).start()
            out_cps.append(store_out(
                2 + 2 * t, agcw_comm[t + 1].astype(jnp.float32),
                (my - t) % N_DEV, 0))
            out_cps.append(store_out(
                3 + 2 * t, agccw_comm[t + 1].astype(jnp.float32),
                (my + t) % N_DEV, nh))

        for cp in out_cps:
            cp.wait()
        for t in range(N_DEV - 1):
            ag_rdma(t, agcw_comm, agcw_send, agcw_recv, right).wait_send()
            ag_rdma(t, agccw_comm, agccw_send, agccw_recv, left).wait_send()

    return pl.pallas_call(
        body,
        out_shape=jax.ShapeDtypeStruct((m, n), jnp.float32),
        in_specs=[
            pl.BlockSpec(memory_space=pl.ANY),
            pl.BlockSpec(memory_space=pl.ANY),
        ],
        out_specs=pl.BlockSpec(memory_space=pl.ANY),
        scratch_shapes=[
            pltpu.VMEM((m, k), jnp.float32),
            pltpu.VMEM((k, n), jnp.float32),
            pltpu.VMEM((2, mc, nh), jnp.bfloat16),
            pltpu.VMEM((2, mc, nh), jnp.bfloat16),
            pltpu.VMEM((N_DEV, mc, nh), jnp.bfloat16),
            pltpu.VMEM((N_DEV, mc, nh), jnp.bfloat16),
            pltpu.VMEM((8, mc, nh), jnp.float32),
            pltpu.SemaphoreType.DMA((2,)),
            pltpu.SemaphoreType.DMA((2,)),
            pltpu.SemaphoreType.DMA((2,)),
            pltpu.SemaphoreType.DMA((2,)),
            pltpu.SemaphoreType.DMA((N_DEV - 1,)),
            pltpu.SemaphoreType.DMA((N_DEV - 1,)),
            pltpu.SemaphoreType.DMA((N_DEV - 1,)),
            pltpu.SemaphoreType.DMA((N_DEV - 1,)),
            pltpu.SemaphoreType.DMA((8,)),
            pltpu.SemaphoreType.DMA((N_DEV + 2,)),
        ],
        compiler_params=pltpu.CompilerParams(
            collective_id=0,
            vmem_limit_bytes=100 * 1024 * 1024,
        ),
    )(A, B)


# device time: 92227 ns/iter; 1.1968x vs baseline; 1.1387x over previous
import jax
import jax.numpy as jnp
from jax import lax
from jax.experimental import pallas as pl
from jax.experimental.pallas import tpu as pltpu

N_DEV = 4
N_LANE = 4


def _gelu(z):
    return 0.5 * z * (1.0 + jnp.tanh(0.7978845608 * (z + 0.044715 * z * z * z)))


def kernel(A, B):
    m, k = A.shape
    _, n = B.shape
    mc = m // N_DEV
    nq = n // N_LANE

    def body(a_hbm, b_hbm, out_ref,
             a_v, b_v, rs_comm, ag_comm, stage,
             rs_send, rs_recv, ag_send, ag_recv, out_sems, in_sems):
        my = lax.axis_index("i")
        left = (my + N_DEV - 1) % N_DEV
        right = (my + 1) % N_DEV

        lane_col = [0, nq, 2 * nq, 3 * nq]
        lane_cw = [True, True, False, False]

        def lane_dev(l):
            return right if lane_cw[l] else left

        def lane_chunk(l, s):
            if lane_cw[l]:
                return (my - s - 1) % N_DEV
            return (my + s + 1) % N_DEV

        def load_a(d):
            c = (my + d) % N_DEV
            cp = pltpu.make_async_copy(
                a_hbm.at[pl.ds(c * mc, mc), :],
                a_v.at[pl.ds(c * mc, mc), :],
                in_sems.at[d],
            )
            cp.start()
            return cp

        def load_b(q):
            cp = pltpu.make_async_copy(
                b_hbm.at[:, pl.ds(q * nq, nq)],
                b_v.at[:, pl.ds(q * nq, nq)],
                in_sems.at[N_DEV + q],
            )
            cp.start()
            return cp

        ld_a0 = load_a(0)
        lane_order = [0, 2, 1, 3]
        ld_b = {l: load_b(l) for l in lane_order}

        barrier_sem = pltpu.get_barrier_semaphore()
        for nbr in (left, right):
            pl.semaphore_signal(
                barrier_sem, inc=1,
                device_id=(nbr,), device_id_type=pl.DeviceIdType.MESH,
            )
        pl.semaphore_wait(barrier_sem, 2)

        def partial(c, l):
            return jnp.dot(
                a_v[pl.ds(c * mc, mc), :], b_v[:, pl.ds(lane_col[l], nq)],
                preferred_element_type=jnp.float32,
            )

        def rs_rdma(l, s):
            ss = s % 2
            rr = (s + 1) % 2
            return pltpu.make_async_remote_copy(
                src_ref=rs_comm.at[l, ss], dst_ref=rs_comm.at[l, rr],
                send_sem=rs_send.at[l, ss], recv_sem=rs_recv.at[l, rr],
                device_id=(lane_dev(l),), device_id_type=pl.DeviceIdType.MESH,
            )

        def ag_rdma(l, t):
            return pltpu.make_async_remote_copy(
                src_ref=ag_comm.at[l, t], dst_ref=ag_comm.at[l, t + 1],
                send_sem=ag_send.at[l, t], recv_sem=ag_recv.at[l, t],
                device_id=(lane_dev(l),), device_id_type=pl.DeviceIdType.MESH,
            )

        def store_out(slot, vals, c, l):
            stage[slot] = vals
            cp = pltpu.make_async_copy(
                stage.at[slot],
                out_ref.at[pl.ds(c * mc, mc), pl.ds(lane_col[l], nq)],
                out_sems.at[slot],
            )
            cp.start()
            return cp

        ld_a0.wait()
        rdmas = {}
        for l in lane_order:
            ld_b[l].wait()
            rs_comm[l, 0] = partial(my, l).astype(jnp.bfloat16)
            r = rs_rdma(l, 0)
            r.start()
            rdmas[l] = r
        ld_rest = {d: load_a(d) for d in range(1, N_DEV)}

        for s in range(N_DEV - 1):
            rr = (s + 1) % 2
            for d in sorted({s + 1, N_DEV - s - 1}):
                cp = ld_rest.pop(d, None)
                if cp is not None:
                    cp.wait()
            for l in lane_order:
                p = partial(lane_chunk(l, s), l)
                rdmas[l].wait()
                rs_comm[l, rr] = (
                    rs_comm[l, rr].astype(jnp.float32) + p
                ).astype(jnp.bfloat16)
                if s < N_DEV - 2:
                    r = rs_rdma(l, s + 1)
                    r.start()
                    rdmas[l] = r

        last = (N_DEV - 1) % 2
        own = [(my + 1) % N_DEV if lane_cw[l] else (my + N_DEV - 1) % N_DEV
               for l in range(N_LANE)]
        out_cps = []
        g_lane = {}
        for l in lane_order:
            g = _gelu(rs_comm[l, last].astype(jnp.float32))
            ag_comm[l, 0] = g.astype(jnp.bfloat16)
            ag_rdma(l, 0).start()
            g_lane[l] = g
        for l in lane_order:
            out_cps.append(store_out(l, g_lane[l], own[l], l))

        for t in range(N_DEV - 1):
            for l in lane_order:
                ag_rdma(l, t).wait_recv()
                if t < N_DEV - 2:
                    ag_rdma(l, t + 1).start()
                c = (my - t) % N_DEV if lane_cw[l] else (my + t) % N_DEV
                out_cps.append(store_out(
                    N_LANE * (t + 1) + l,
                    ag_comm[l, t + 1].astype(jnp.float32), c, l))

        for cp in out_cps:
            cp.wait()
        for t in range(N_DEV - 1):
            for l in range(N_LANE):
                ag_rdma(l, t).wait_send()

    return pl.pallas_call(
        body,
        out_shape=jax.ShapeDtypeStruct((m, n), jnp.float32),
        in_specs=[
            pl.BlockSpec(memory_space=pl.ANY),
            pl.BlockSpec(memory_space=pl.ANY),
        ],
        out_specs=pl.BlockSpec(memory_space=pl.ANY),
        scratch_shapes=[
            pltpu.VMEM((m, k), jnp.float32),
            pltpu.VMEM((k, n), jnp.float32),
            pltpu.VMEM((N_LANE, 2, mc, nq), jnp.bfloat16),
            pltpu.VMEM((N_LANE, N_DEV, mc, nq), jnp.bfloat16),
            pltpu.VMEM((N_LANE * N_DEV, mc, nq), jnp.float32),
            pltpu.SemaphoreType.DMA((N_LANE, 2)),
            pltpu.SemaphoreType.DMA((N_LANE, 2)),
            pltpu.SemaphoreType.DMA((N_LANE, N_DEV - 1)),
            pltpu.SemaphoreType.DMA((N_LANE, N_DEV - 1)),
            pltpu.SemaphoreType.DMA((N_LANE * N_DEV,)),
            pltpu.SemaphoreType.DMA((N_DEV + N_LANE,)),
        ],
        compiler_params=pltpu.CompilerParams(
            collective_id=0,
            vmem_limit_bytes=100 * 1024 * 1024,
        ),
    )(A, B)


# device time: 86759 ns/iter; 1.2722x vs baseline; 1.0630x over previous
import jax
import jax.numpy as jnp
from jax import lax
from jax.experimental import pallas as pl
from jax.experimental.pallas import tpu as pltpu

N_DEV = 4
N_LANE = 4


def _gelu(z):
    return 0.5 * z * (1.0 + jnp.tanh(0.7978845608 * (z + 0.044715 * z * z * z)))


def kernel(A, B):
    m, k = A.shape
    _, n = B.shape
    mc = m // N_DEV
    nq = n // N_LANE

    def body(a_hbm, b_hbm, out_ref,
             a_v, b_v, rs_comm, ag_comm,
             rs_send, rs_recv, ag_send, ag_recv, out_sems, in_sems):
        my = lax.axis_index("i")
        left = (my + N_DEV - 1) % N_DEV
        right = (my + 1) % N_DEV

        lane_col = [0, nq, 2 * nq, 3 * nq]
        lane_cw = [True, True, False, False]

        def lane_dev(l):
            return right if lane_cw[l] else left

        def lane_chunk(l, s):
            if lane_cw[l]:
                return (my - s - 1) % N_DEV
            return (my + s + 1) % N_DEV

        def load_a(d):
            c = (my + d) % N_DEV
            cp = pltpu.make_async_copy(
                a_hbm.at[pl.ds(c * mc, mc), :],
                a_v.at[pl.ds(c * mc, mc), :],
                in_sems.at[d],
            )
            cp.start()
            return cp

        def load_b(q):
            cp = pltpu.make_async_copy(
                b_hbm.at[:, pl.ds(q * nq, nq)],
                b_v.at[:, pl.ds(q * nq, nq)],
                in_sems.at[N_DEV + q],
            )
            cp.start()
            return cp

        ld_a0 = load_a(0)
        lane_order = [0, 2, 1, 3]
        ld_b = {l: load_b(l) for l in lane_order}

        barrier_sem = pltpu.get_barrier_semaphore()
        for nbr in (left, right):
            pl.semaphore_signal(
                barrier_sem, inc=1,
                device_id=(nbr,), device_id_type=pl.DeviceIdType.MESH,
            )
        pl.semaphore_wait(barrier_sem, 2)

        def partial(c, l):
            return jnp.dot(
                a_v[pl.ds(c * mc, mc), :], b_v[:, pl.ds(lane_col[l], nq)],
                preferred_element_type=jnp.float32,
            )

        def rs_rdma(l, s):
            ss = s % 2
            rr = (s + 1) % 2
            return pltpu.make_async_remote_copy(
                src_ref=rs_comm.at[l, ss], dst_ref=rs_comm.at[l, rr],
                send_sem=rs_send.at[l, ss], recv_sem=rs_recv.at[l, rr],
                device_id=(lane_dev(l),), device_id_type=pl.DeviceIdType.MESH,
            )

        def ag_rdma(l, t):
            return pltpu.make_async_remote_copy(
                src_ref=ag_comm.at[l, t], dst_ref=ag_comm.at[l, t + 1],
                send_sem=ag_send.at[l, t], recv_sem=ag_recv.at[l, t],
                device_id=(lane_dev(l),), device_id_type=pl.DeviceIdType.MESH,
            )

        def store_out(slot, l, t, c):
            cp = pltpu.make_async_copy(
                ag_comm.at[l, t],
                out_ref.at[pl.ds(c * mc, mc), pl.ds(lane_col[l], nq)],
                out_sems.at[slot],
            )
            cp.start()
            return cp

        ld_a0.wait()
        rdmas = {}
        for l in lane_order:
            ld_b[l].wait()
            rs_comm[l, 0] = partial(my, l).astype(jnp.bfloat16)
            r = rs_rdma(l, 0)
            r.start()
            rdmas[l] = r
        ld_rest = {d: load_a(d) for d in range(1, N_DEV)}

        for s in range(N_DEV - 1):
            rr = (s + 1) % 2
            for d in sorted({s + 1, N_DEV - s - 1}):
                cp = ld_rest.pop(d, None)
                if cp is not None:
                    cp.wait()
            for l in lane_order:
                p = partial(lane_chunk(l, s), l)
                rdmas[l].wait()
                rs_comm[l, rr] = (
                    rs_comm[l, rr].astype(jnp.float32) + p
                ).astype(jnp.bfloat16)
                if s < N_DEV - 2:
                    r = rs_rdma(l, s + 1)
                    r.start()
                    rdmas[l] = r

        last = (N_DEV - 1) % 2
        own = [(my + 1) % N_DEV if lane_cw[l] else (my + N_DEV - 1) % N_DEV
               for l in range(N_LANE)]
        out_cps = []
        for l in lane_order:
            g = _gelu(rs_comm[l, last].astype(jnp.float32))
            ag_comm[l, 0] = g.astype(jnp.bfloat16)
            ag_rdma(l, 0).start()
        for l in lane_order:
            out_cps.append(store_out(l, l, 0, own[l]))

        for t in range(N_DEV - 1):
            for l in lane_order:
                ag_rdma(l, t).wait_recv()
                if t < N_DEV - 2:
                    ag_rdma(l, t + 1).start()
                c = (my - t) % N_DEV if lane_cw[l] else (my + t) % N_DEV
                out_cps.append(store_out(
                    N_LANE * (t + 1) + l, l, t + 1, c))

        for cp in out_cps:
            cp.wait()
        for t in range(N_DEV - 1):
            for l in range(N_LANE):
                ag_rdma(l, t).wait_send()

    return pl.pallas_call(
        body,
        out_shape=jax.ShapeDtypeStruct((m, n), jnp.bfloat16),
        in_specs=[
            pl.BlockSpec(memory_space=pl.ANY),
            pl.BlockSpec(memory_space=pl.ANY),
        ],
        out_specs=pl.BlockSpec(memory_space=pl.ANY),
        scratch_shapes=[
            pltpu.VMEM((m, k), jnp.float32),
            pltpu.VMEM((k, n), jnp.float32),
            pltpu.VMEM((N_LANE, 2, mc, nq), jnp.bfloat16),
            pltpu.VMEM((N_LANE, N_DEV, mc, nq), jnp.bfloat16),
            pltpu.SemaphoreType.DMA((N_LANE, 2)),
            pltpu.SemaphoreType.DMA((N_LANE, 2)),
            pltpu.SemaphoreType.DMA((N_LANE, N_DEV - 1)),
            pltpu.SemaphoreType.DMA((N_LANE, N_DEV - 1)),
            pltpu.SemaphoreType.DMA((N_LANE * N_DEV,)),
            pltpu.SemaphoreType.DMA((N_DEV + N_LANE,)),
        ],
        compiler_params=pltpu.CompilerParams(
            collective_id=0,
            vmem_limit_bytes=100 * 1024 * 1024,
        ),
    )(A, B)


# device time: 86385 ns/iter; 1.2777x vs baseline; 1.0043x over previous
import jax
import jax.numpy as jnp
from jax import lax
from jax.experimental import pallas as pl
from jax.experimental.pallas import tpu as pltpu

N_DEV = 4
N_LANE = 8


def _gelu(z):
    return 0.5 * z * (1.0 + jnp.tanh(0.7978845608 * (z + 0.044715 * z * z * z)))


def kernel(A, B):
    m, k = A.shape
    _, n = B.shape
    mc = m // N_DEV
    nq = n // N_LANE

    def body(a_hbm, b_hbm, out_ref,
             a_v, b_v, rs_comm, ag_comm,
             rs_send, rs_recv, ag_send, ag_recv, out_sems, in_sems):
        my = lax.axis_index("i")
        left = (my + N_DEV - 1) % N_DEV
        right = (my + 1) % N_DEV

        lane_col = [l * nq for l in range(N_LANE)]
        lane_cw = [l < N_LANE // 2 for l in range(N_LANE)]

        def lane_dev(l):
            return right if lane_cw[l] else left

        def lane_chunk(l, s):
            if lane_cw[l]:
                return (my - s - 1) % N_DEV
            return (my + s + 1) % N_DEV

        def load_a(d):
            c = (my + d) % N_DEV
            cp = pltpu.make_async_copy(
                a_hbm.at[pl.ds(c * mc, mc), :],
                a_v.at[pl.ds(c * mc, mc), :],
                in_sems.at[d],
            )
            cp.start()
            return cp

        def load_b(q):
            cp = pltpu.make_async_copy(
                b_hbm.at[:, pl.ds(q * nq, nq)],
                b_v.at[:, pl.ds(q * nq, nq)],
                in_sems.at[N_DEV + q],
            )
            cp.start()
            return cp

        ld_a0 = load_a(0)
        lane_order = [l // 2 + (l % 2) * (N_LANE // 2) for l in range(N_LANE)]
        ld_b = {l: load_b(l) for l in lane_order}

        barrier_sem = pltpu.get_barrier_semaphore()
        for nbr in (left, right):
            pl.semaphore_signal(
                barrier_sem, inc=1,
                device_id=(nbr,), device_id_type=pl.DeviceIdType.MESH,
            )
        pl.semaphore_wait(barrier_sem, 2)

        def partial(c, l):
            return jnp.dot(
                a_v[pl.ds(c * mc, mc), :], b_v[:, pl.ds(lane_col[l], nq)],
                preferred_element_type=jnp.float32,
            )

        def rs_rdma(l, s):
            ss = s % 2
            rr = (s + 1) % 2
            return pltpu.make_async_remote_copy(
                src_ref=rs_comm.at[l, ss], dst_ref=rs_comm.at[l, rr],
                send_sem=rs_send.at[l, ss], recv_sem=rs_recv.at[l, rr],
                device_id=(lane_dev(l),), device_id_type=pl.DeviceIdType.MESH,
            )

        def ag_rdma(l, t):
            return pltpu.make_async_remote_copy(
                src_ref=ag_comm.at[l, t], dst_ref=ag_comm.at[l, t + 1],
                send_sem=ag_send.at[l, t], recv_sem=ag_recv.at[l, t],
                device_id=(lane_dev(l),), device_id_type=pl.DeviceIdType.MESH,
            )

        def store_out(slot, l, t, c):
            cp = pltpu.make_async_copy(
                ag_comm.at[l, t],
                out_ref.at[pl.ds(c * mc, mc), pl.ds(lane_col[l], nq)],
                out_sems.at[slot],
            )
            cp.start()
            return cp

        ld_a0.wait()
        rdmas = {}
        for l in lane_order:
            ld_b[l].wait()
            rs_comm[l, 0] = partial(my, l).astype(jnp.bfloat16)
            r = rs_rdma(l, 0)
            r.start()
            rdmas[l] = r
        ld_rest = {d: load_a(d) for d in range(1, N_DEV)}

        for s in range(N_DEV - 1):
            rr = (s + 1) % 2
            for d in sorted({s + 1, N_DEV - s - 1}):
                cp = ld_rest.pop(d, None)
                if cp is not None:
                    cp.wait()
            for l in lane_order:
                p = partial(lane_chunk(l, s), l)
                rdmas[l].wait()
                rs_comm[l, rr] = (
                    rs_comm[l, rr].astype(jnp.float32) + p
                ).astype(jnp.bfloat16)
                if s < N_DEV - 2:
                    r = rs_rdma(l, s + 1)
                    r.start()
                    rdmas[l] = r

        last = (N_DEV - 1) % 2
        own = [(my + 1) % N_DEV if lane_cw[l] else (my + N_DEV - 1) % N_DEV
               for l in range(N_LANE)]
        out_cps = []
        for l in lane_order:
            g = _gelu(rs_comm[l, last].astype(jnp.float32))
            ag_comm[l, 0] = g.astype(jnp.bfloat16)
            ag_rdma(l, 0).start()
        for l in lane_order:
            out_cps.append(store_out(l, l, 0, own[l]))

        for t in range(N_DEV - 1):
            for l in lane_order:
                ag_rdma(l, t).wait_recv()
                if t < N_DEV - 2:
                    ag_rdma(l, t + 1).start()
                c = (my - t) % N_DEV if lane_cw[l] else (my + t) % N_DEV
                out_cps.append(store_out(
                    N_LANE * (t + 1) + l, l, t + 1, c))

        for cp in out_cps:
            cp.wait()
        for t in range(N_DEV - 1):
            for l in range(N_LANE):
                ag_rdma(l, t).wait_send()

    return pl.pallas_call(
        body,
        out_shape=jax.ShapeDtypeStruct((m, n), jnp.bfloat16),
        in_specs=[
            pl.BlockSpec(memory_space=pl.ANY),
            pl.BlockSpec(memory_space=pl.ANY),
        ],
        out_specs=pl.BlockSpec(memory_space=pl.ANY),
        scratch_shapes=[
            pltpu.VMEM((m, k), jnp.float32),
            pltpu.VMEM((k, n), jnp.float32),
            pltpu.VMEM((N_LANE, 2, mc, nq), jnp.bfloat16),
            pltpu.VMEM((N_LANE, N_DEV, mc, nq), jnp.bfloat16),
            pltpu.SemaphoreType.DMA((N_LANE, 2)),
            pltpu.SemaphoreType.DMA((N_LANE, 2)),
            pltpu.SemaphoreType.DMA((N_LANE, N_DEV - 1)),
            pltpu.SemaphoreType.DMA((N_LANE, N_DEV - 1)),
            pltpu.SemaphoreType.DMA((N_LANE * N_DEV,)),
            pltpu.SemaphoreType.DMA((N_DEV + N_LANE,)),
        ],
        compiler_params=pltpu.CompilerParams(
            collective_id=0,
            vmem_limit_bytes=100 * 1024 * 1024,
        ),
    )(A, B)
